# Initial kernel scaffold; baseline (speedup 1.0000x reference)
import jax
import jax.numpy as jnp
from jax.experimental import pallas as pl


def kernel(category_id):
    raise NotImplementedError("write your pallas kernel here")


# speedup vs baseline: 24.4067x; 24.4067x over previous
import functools

import jax
import jax.numpy as jnp
from jax import lax
from jax.experimental import pallas as pl
from jax.experimental.pallas import tpu as pltpu
from jax.experimental.pallas import tpu_sc as plsc

NUM_CLASSES = 1203
C_PAD = 1280
NC, NS, L = 2, 16, 16
NW = NC * NS


def _sc_hist_body(cat_hbm, out_hbm, data, marker, hist, local,
                  *, rows_per_lane, ann):
    wid = lax.axis_index("s") * NC + lax.axis_index("c")
    per_tile = rows_per_lane * L * ann
    pltpu.sync_copy(cat_hbm.at[pl.ds(wid * per_tile, per_tile)], data)

    iota = lax.iota(jnp.int32, 16)
    lanebase = iota * C_PAD
    datbase = iota * (rows_per_lane * ann)
    ones = jnp.ones((16,), jnp.int32)

    def init_body(i, carry):
        marker[pl.ds(i * 16, 16)] = jnp.full((16,), -1, jnp.int32)
        hist[pl.ds(i * 16, 16)] = jnp.zeros((16,), jnp.int32)
        return carry

    lax.fori_loop(0, L * C_PAD // 16, init_body, 0)

    def row_body(r, carry):
        stamp = jnp.full((16,), r, jnp.int32)
        for j in range(ann):
            didx = datbase + (r * ann + j)
            c = plsc.load_gather(data, [didx])
            midx = lanebase + c
            old = plsc.load_gather(marker, [midx])
            fresh = old != stamp
            plsc.addupdate_scatter(hist, [midx], ones, mask=fresh)
            plsc.store_scatter(marker, [midx], stamp)
        return carry

    lax.fori_loop(0, rows_per_lane, row_body, 0)

    def red_body(jj, carry):
        acc = hist[pl.ds(jj * 16, 16)]
        for l in range(1, L):
            acc = acc + hist[pl.ds(l * C_PAD + jj * 16, 16)]
        local[pl.ds(jj * 16, 16)] = acc
        return carry

    lax.fori_loop(0, C_PAD // 16, red_body, 0)
    pltpu.sync_copy(local, out_hbm.at[wid])


def _sc_hist(cat_flat, rows_per_lane, ann):
    mesh = plsc.VectorSubcoreMesh(
        core_axis_name="c", subcore_axis_name="s",
        num_cores=NC, num_subcores=NS)
    per_tile = rows_per_lane * L * ann
    run = pl.kernel(
        functools.partial(_sc_hist_body, rows_per_lane=rows_per_lane,
                          ann=ann),
        out_type=jax.ShapeDtypeStruct((NW, C_PAD), jnp.int32),
        mesh=mesh,
        scratch_types=[
            pltpu.VMEM((per_tile,), jnp.int32),
            pltpu.VMEM((L * C_PAD,), jnp.int32),
            pltpu.VMEM((L * C_PAD,), jnp.int32),
            pltpu.VMEM((C_PAD,), jnp.int32),
        ],
        compiler_params=pltpu.CompilerParams(needs_layout_passes=False),
    )
    return run(cat_flat)


def _tc_idf_body(n_rows, counts_ref, out_ref):
    df = jnp.sum(counts_ref[...], axis=0, keepdims=True)
    df = df.astype(jnp.float32) + 1.0
    out_ref[...] = jnp.log((n_rows + 1) / df) + 1.0


def _tc_idf(counts, n_rows):
    return pl.pallas_call(
        functools.partial(_tc_idf_body, n_rows),
        out_shape=jax.ShapeDtypeStruct((1, C_PAD), jnp.float32),
    )(counts)


@jax.jit
def kernel(category_id):
    n_rows, ann = category_id.shape
    rows_per_lane = n_rows // (NW * L)
    cat_flat = category_id.reshape(-1)
    counts = _sc_hist(cat_flat, rows_per_lane, ann)
    weights = _tc_idf(counts, n_rows)
    return weights[0, :NUM_CLASSES]


# speedup vs baseline: 30.0084x; 1.2295x over previous
import functools

import jax
import jax.numpy as jnp
from jax import lax
from jax.experimental import pallas as pl
from jax.experimental.pallas import tpu as pltpu
from jax.experimental.pallas import tpu_sc as plsc

NUM_CLASSES = 1203
C_PAD = 1280
NC, NS, L = 2, 16, 16
NW = NC * NS


def _sc_hist_body(cat_hbm, out_hbm, data, marker_a, marker_b, hist, sem,
                  *, rows_per_lane, ann):
    wid = lax.axis_index("s") * NC + lax.axis_index("c")
    per_lane = rows_per_lane * ann
    per_tile = per_lane * L
    copy = pltpu.async_copy(
        cat_hbm.at[pl.ds(wid * per_tile, per_tile)], data, sem)

    iota = lax.iota(jnp.int32, 16)
    lanebase = iota * C_PAD
    ones = jnp.ones((16,), jnp.int32)
    neg1 = jnp.full((16,), -1, jnp.int32)
    zero = jnp.zeros((16,), jnp.int32)

    def init_body(i, carry):
        for u in range(8):
            marker_a[pl.ds(i * 128 + u * 16, 16)] = neg1
            marker_b[pl.ds(i * 128 + u * 16, 16)] = neg1
        return carry

    lax.fori_loop(0, L * C_PAD // 128, init_body, 0)

    def hinit_body(i, carry):
        for u in range(8):
            hist[pl.ds(i * 128 + u * 16, 16)] = zero
        return carry

    lax.fori_loop(0, C_PAD // 128, hinit_body, 0)
    copy.wait()

    half = rows_per_lane // 2

    def row_body(r, carry):
        stamp = jnp.full((16,), r, jnp.int32)
        for j in range(ann):
            off = r * ann + j
            didx_a = iota * per_lane + off
            didx_b = didx_a + half * ann
            ca = plsc.load_gather(data, [didx_a])
            cb = plsc.load_gather(data, [didx_b])
            midx_a = lanebase + ca
            midx_b = lanebase + cb
            olda = plsc.load_gather(marker_a, [midx_a])
            oldb = plsc.load_gather(marker_b, [midx_b])
            fresh_a = olda != stamp
            fresh_b = oldb != stamp
            plsc.addupdate_scatter(hist, [ca], ones, mask=fresh_a)
            plsc.addupdate_scatter(hist, [cb], ones, mask=fresh_b)
            plsc.store_scatter(marker_a, [midx_a], stamp)
            plsc.store_scatter(marker_b, [midx_b], stamp)
        return carry

    lax.fori_loop(0, half, row_body, 0)
    pltpu.sync_copy(hist, out_hbm.at[wid])


def _sc_hist(cat_flat, rows_per_lane, ann):
    mesh = plsc.VectorSubcoreMesh(
        core_axis_name="c", subcore_axis_name="s",
        num_cores=NC, num_subcores=NS)
    per_tile = rows_per_lane * L * ann
    run = pl.kernel(
        functools.partial(_sc_hist_body, rows_per_lane=rows_per_lane,
                          ann=ann),
        out_type=jax.ShapeDtypeStruct((NW, C_PAD), jnp.int32),
        mesh=mesh,
        scratch_types=[
            pltpu.VMEM((per_tile,), jnp.int32),
            pltpu.VMEM((L * C_PAD,), jnp.int32),
            pltpu.VMEM((L * C_PAD,), jnp.int32),
            pltpu.VMEM((C_PAD,), jnp.int32),
            pltpu.SemaphoreType.DMA,
        ],
        compiler_params=pltpu.CompilerParams(needs_layout_passes=False),
    )
    return run(cat_flat)


def _tc_idf_body(n_rows, counts_ref, out_ref):
    df = jnp.sum(counts_ref[...], axis=0, keepdims=True)
    df = df.astype(jnp.float32) + 1.0
    out_ref[...] = jnp.log((n_rows + 1) / df) + 1.0


def _tc_idf(counts, n_rows):
    return pl.pallas_call(
        functools.partial(_tc_idf_body, n_rows),
        out_shape=jax.ShapeDtypeStruct((1, C_PAD), jnp.float32),
    )(counts)


@jax.jit
def kernel(category_id):
    n_rows, ann = category_id.shape
    rows_per_lane = n_rows // (NW * L)
    cat_flat = category_id.reshape(-1)
    counts = _sc_hist(cat_flat, rows_per_lane, ann)
    weights = _tc_idf(counts, n_rows)
    return weights[0, :NUM_CLASSES]
